# baseline (speedup 1.0000x reference)
import jax
import jax.numpy as jnp
from jax.experimental import pallas as pl
from jax.experimental.pallas import tpu as pltpu

_NUM_ATTRIB = 85
_AW = (30.0, 62.0, 59.0)
_AH = (61.0, 45.0, 119.0)
_ROWS_PER_IMG = 19200
_BLK_ROWS = 960
_SC_IMGS = 2
_TC_IMGS = 6


def _floordiv_f32(x, d):
    return jnp.floor((x + 0.5) * (1.0 / d))


def _decode(v, i):
    a = jax.lax.broadcasted_iota(jnp.int32, (1, _NUM_ATTRIB), 1)
    n = jnp.float32(i * _BLK_ROWS) + jax.lax.broadcasted_iota(
        jnp.int32, (_BLK_ROWS, 1), 0).astype(jnp.float32)
    pos = _floordiv_f32(n, 3.0)
    j = n - 3.0 * pos
    gy = _floordiv_f32(pos, 80.0)
    gx = pos - 80.0 * gy
    is_sig = a >= 4
    e = jnp.exp(jnp.where(is_sig, -v, v))
    sig = 1.0 / (1.0 + e)
    wsel = jnp.where(j == 0.0, _AW[0], jnp.where(j == 1.0, _AW[1], _AW[2]))
    hsel = jnp.where(j == 0.0, _AH[0], jnp.where(j == 1.0, _AH[1], _AH[2]))
    dim = jnp.where(a == 2, wsel, hsel)
    g = jnp.where(a == 0, gx, gy)
    lin = jnp.where((a == 2) | (a == 3), e * dim, (v + g) * 16.0)
    return jnp.where(is_sig, sig, lin)


def _body1(x_ref, o_ref):
    o_ref[0] = _decode(x_ref[0], pl.program_id(1))


def _body2(scale_ref, x_ref, prev_ref, o_ref):
    del prev_ref
    o_ref[0] = _decode(x_ref[0] * scale_ref[0, 0], pl.program_id(1))


def kernel(pred_map, num_imgs, level_idx):
    del level_idx
    ni = pred_map.shape[0]
    scale = jnp.asarray(num_imgs, jnp.float32) / ni
    y0 = jax.lax.optimization_barrier(pred_map[:_SC_IMGS]).reshape(
        _SC_IMGS, _ROWS_PER_IMG, _NUM_ATTRIB)
    y1 = pred_map[_SC_IMGS:].reshape(
        _TC_IMGS, _ROWS_PER_IMG, _NUM_ATTRIB) * scale
    blk = (1, _BLK_ROWS, _NUM_ATTRIB)
    out_sd = jax.ShapeDtypeStruct((ni, _ROWS_PER_IMG, _NUM_ATTRIB),
                                  jnp.float32)
    o1 = pl.pallas_call(
        _body1,
        grid=(_TC_IMGS, _ROWS_PER_IMG // _BLK_ROWS),
        in_specs=[pl.BlockSpec(blk, lambda b, i: (b, i, 0))],
        out_specs=pl.BlockSpec(blk, lambda b, i: (b + _SC_IMGS, i, 0)),
        out_shape=out_sd,
    )(y1)
    o2 = pl.pallas_call(
        _body2,
        grid=(_SC_IMGS, _ROWS_PER_IMG // _BLK_ROWS),
        in_specs=[
            pl.BlockSpec(memory_space=pltpu.SMEM),
            pl.BlockSpec(blk, lambda b, i: (b, i, 0)),
            pl.BlockSpec(memory_space=pl.ANY),
        ],
        out_specs=pl.BlockSpec(blk, lambda b, i: (b, i, 0)),
        out_shape=out_sd,
        input_output_aliases={2: 0},
    )(scale.reshape(1, 1), y0, o1)
    return o2


# speedup vs baseline: 1.3860x; 1.3860x over previous
import jax
import jax.numpy as jnp
from jax.experimental import pallas as pl
from jax.experimental.pallas import tpu as pltpu

_NUM_ATTRIB = 85
_AW = (30.0, 62.0, 59.0)
_AH = (61.0, 45.0, 119.0)
_ROWS_PER_IMG = 19200
_BLK_ROWS = 960
_SC_IMGS = 4
_TC_IMGS = 4


def _floordiv_f32(x, d):
    return jnp.floor((x + 0.5) * (1.0 / d))


def _decode(v, i):
    a = jax.lax.broadcasted_iota(jnp.int32, (1, _NUM_ATTRIB), 1)
    n = jnp.float32(i * _BLK_ROWS) + jax.lax.broadcasted_iota(
        jnp.int32, (_BLK_ROWS, 1), 0).astype(jnp.float32)
    pos = _floordiv_f32(n, 3.0)
    j = n - 3.0 * pos
    gy = _floordiv_f32(pos, 80.0)
    gx = pos - 80.0 * gy
    is_sig = a >= 4
    e = jnp.exp(jnp.where(is_sig, -v, v))
    sig = 1.0 / (1.0 + e)
    wsel = jnp.where(j == 0.0, _AW[0], jnp.where(j == 1.0, _AW[1], _AW[2]))
    hsel = jnp.where(j == 0.0, _AH[0], jnp.where(j == 1.0, _AH[1], _AH[2]))
    dim = jnp.where(a == 2, wsel, hsel)
    g = jnp.where(a == 0, gx, gy)
    lin = jnp.where((a == 2) | (a == 3), e * dim, (v + g) * 16.0)
    return jnp.where(is_sig, sig, lin)


def _body1(x_ref, o_ref):
    o_ref[0] = _decode(x_ref[0], pl.program_id(1))


def _body2(scale_ref, x_ref, prev_ref, o_ref):
    del prev_ref
    o_ref[0] = _decode(x_ref[0] * scale_ref[0, 0], pl.program_id(1))


def kernel(pred_map, num_imgs, level_idx):
    del level_idx
    ni = pred_map.shape[0]
    scale = jnp.asarray(num_imgs, jnp.float32) / ni
    y0 = jax.lax.optimization_barrier(pred_map[:_SC_IMGS]).reshape(
        _SC_IMGS, _ROWS_PER_IMG, _NUM_ATTRIB)
    y1 = pred_map[_SC_IMGS:].reshape(
        _TC_IMGS, _ROWS_PER_IMG, _NUM_ATTRIB) * scale
    blk = (1, _BLK_ROWS, _NUM_ATTRIB)
    out_sd = jax.ShapeDtypeStruct((ni, _ROWS_PER_IMG, _NUM_ATTRIB),
                                  jnp.float32)
    o1 = pl.pallas_call(
        _body1,
        grid=(_TC_IMGS, _ROWS_PER_IMG // _BLK_ROWS),
        in_specs=[pl.BlockSpec(blk, lambda b, i: (b, i, 0))],
        out_specs=pl.BlockSpec(blk, lambda b, i: (b + _SC_IMGS, i, 0)),
        out_shape=out_sd,
    )(y1)
    o2 = pl.pallas_call(
        _body2,
        grid=(_SC_IMGS, _ROWS_PER_IMG // _BLK_ROWS),
        in_specs=[
            pl.BlockSpec(memory_space=pltpu.SMEM),
            pl.BlockSpec(blk, lambda b, i: (b, i, 0)),
            pl.BlockSpec(memory_space=pl.ANY),
        ],
        out_specs=pl.BlockSpec(blk, lambda b, i: (b, i, 0)),
        out_shape=out_sd,
        input_output_aliases={2: 0},
    )(scale.reshape(1, 1), y0, o1)
    return o2


# speedup vs baseline: 1.5484x; 1.1171x over previous
import jax
import jax.numpy as jnp
from jax.experimental import pallas as pl
from jax.experimental.pallas import tpu as pltpu

_NUM_ATTRIB = 85
_AW = (30.0, 62.0, 59.0)
_AH = (61.0, 45.0, 119.0)
_ROWS_PER_IMG = 19200
_BLK_ROWS = 2400
_SC_IMGS = 4
_TC_IMGS = 4


def _floordiv_f32(x, d):
    return jnp.floor((x + 0.5) * (1.0 / d))


def _decode(v, i):
    a = jax.lax.broadcasted_iota(jnp.int32, (1, _NUM_ATTRIB), 1)
    n = jnp.float32(i * _BLK_ROWS) + jax.lax.broadcasted_iota(
        jnp.int32, (_BLK_ROWS, 1), 0).astype(jnp.float32)
    pos = _floordiv_f32(n, 3.0)
    j = n - 3.0 * pos
    gy = _floordiv_f32(pos, 80.0)
    gx = pos - 80.0 * gy
    is_sig = a >= 4
    e = jnp.exp(jnp.where(is_sig, -v, v))
    sig = 1.0 / (1.0 + e)
    wsel = jnp.where(j == 0.0, _AW[0], jnp.where(j == 1.0, _AW[1], _AW[2]))
    hsel = jnp.where(j == 0.0, _AH[0], jnp.where(j == 1.0, _AH[1], _AH[2]))
    dim = jnp.where(a == 2, wsel, hsel)
    g = jnp.where(a == 0, gx, gy)
    lin = jnp.where((a == 2) | (a == 3), e * dim, (v + g) * 16.0)
    return jnp.where(is_sig, sig, lin)


def _body1(x_ref, o_ref):
    o_ref[0] = _decode(x_ref[0], pl.program_id(1))


def _body2(scale_ref, x_ref, prev_ref, o_ref):
    del prev_ref
    o_ref[0] = _decode(x_ref[0] * scale_ref[0, 0], pl.program_id(1))


def kernel(pred_map, num_imgs, level_idx):
    del level_idx
    ni = pred_map.shape[0]
    scale = jnp.asarray(num_imgs, jnp.float32) / ni
    y0 = jax.lax.optimization_barrier(pred_map[:_SC_IMGS]).reshape(
        _SC_IMGS, _ROWS_PER_IMG, _NUM_ATTRIB)
    y1 = pred_map[_SC_IMGS:].reshape(
        _TC_IMGS, _ROWS_PER_IMG, _NUM_ATTRIB) * scale
    blk = (1, _BLK_ROWS, _NUM_ATTRIB)
    out_sd = jax.ShapeDtypeStruct((ni, _ROWS_PER_IMG, _NUM_ATTRIB),
                                  jnp.float32)
    o1 = pl.pallas_call(
        _body1,
        grid=(_TC_IMGS, _ROWS_PER_IMG // _BLK_ROWS),
        in_specs=[pl.BlockSpec(blk, lambda b, i: (b, i, 0))],
        out_specs=pl.BlockSpec(blk, lambda b, i: (b + _SC_IMGS, i, 0)),
        out_shape=out_sd,
    )(y1)
    o2 = pl.pallas_call(
        _body2,
        grid=(_SC_IMGS, _ROWS_PER_IMG // _BLK_ROWS),
        in_specs=[
            pl.BlockSpec(memory_space=pltpu.SMEM),
            pl.BlockSpec(blk, lambda b, i: (b, i, 0)),
            pl.BlockSpec(memory_space=pl.ANY),
        ],
        out_specs=pl.BlockSpec(blk, lambda b, i: (b, i, 0)),
        out_shape=out_sd,
        input_output_aliases={2: 0},
    )(scale.reshape(1, 1), y0, o1)
    return o2


# speedup vs baseline: 1.6029x; 1.0352x over previous
import jax
import jax.numpy as jnp
from jax.experimental import pallas as pl
from jax.experimental.pallas import tpu as pltpu

_NUM_ATTRIB = 85
_AW = (30.0, 62.0, 59.0)
_AH = (61.0, 45.0, 119.0)
_ROWS_PER_IMG = 19200
_BLK_ROWS = 4800
_SC_IMGS = 4
_TC_IMGS = 4


def _floordiv_f32(x, d):
    return jnp.floor((x + 0.5) * (1.0 / d))


def _decode(v, i):
    a = jax.lax.broadcasted_iota(jnp.int32, (1, _NUM_ATTRIB), 1)
    n = jnp.float32(i * _BLK_ROWS) + jax.lax.broadcasted_iota(
        jnp.int32, (_BLK_ROWS, 1), 0).astype(jnp.float32)
    pos = _floordiv_f32(n, 3.0)
    j = n - 3.0 * pos
    gy = _floordiv_f32(pos, 80.0)
    gx = pos - 80.0 * gy
    is_sig = a >= 4
    e = jnp.exp(jnp.where(is_sig, -v, v))
    sig = 1.0 / (1.0 + e)
    wsel = jnp.where(j == 0.0, _AW[0], jnp.where(j == 1.0, _AW[1], _AW[2]))
    hsel = jnp.where(j == 0.0, _AH[0], jnp.where(j == 1.0, _AH[1], _AH[2]))
    dim = jnp.where(a == 2, wsel, hsel)
    g = jnp.where(a == 0, gx, gy)
    lin = jnp.where((a == 2) | (a == 3), e * dim, (v + g) * 16.0)
    return jnp.where(is_sig, sig, lin)


def _body1(x_ref, o_ref):
    o_ref[0] = _decode(x_ref[0], pl.program_id(1))


def _body2(scale_ref, x_ref, prev_ref, o_ref):
    del prev_ref
    o_ref[0] = _decode(x_ref[0] * scale_ref[0, 0], pl.program_id(1))


def kernel(pred_map, num_imgs, level_idx):
    del level_idx
    ni = pred_map.shape[0]
    scale = jnp.asarray(num_imgs, jnp.float32) / ni
    y0 = jax.lax.optimization_barrier(pred_map[:_SC_IMGS]).reshape(
        _SC_IMGS, _ROWS_PER_IMG, _NUM_ATTRIB)
    y1 = pred_map[_SC_IMGS:].reshape(
        _TC_IMGS, _ROWS_PER_IMG, _NUM_ATTRIB) * scale
    blk = (1, _BLK_ROWS, _NUM_ATTRIB)
    out_sd = jax.ShapeDtypeStruct((ni, _ROWS_PER_IMG, _NUM_ATTRIB),
                                  jnp.float32)
    o1 = pl.pallas_call(
        _body1,
        grid=(_TC_IMGS, _ROWS_PER_IMG // _BLK_ROWS),
        in_specs=[pl.BlockSpec(blk, lambda b, i: (b, i, 0))],
        out_specs=pl.BlockSpec(blk, lambda b, i: (b + _SC_IMGS, i, 0)),
        out_shape=out_sd,
    )(y1)
    o2 = pl.pallas_call(
        _body2,
        grid=(_SC_IMGS, _ROWS_PER_IMG // _BLK_ROWS),
        in_specs=[
            pl.BlockSpec(memory_space=pltpu.SMEM),
            pl.BlockSpec(blk, lambda b, i: (b, i, 0)),
            pl.BlockSpec(memory_space=pl.ANY),
        ],
        out_specs=pl.BlockSpec(blk, lambda b, i: (b, i, 0)),
        out_shape=out_sd,
        input_output_aliases={2: 0},
    )(scale.reshape(1, 1), y0, o1)
    return o2


# speedup vs baseline: 1.6086x; 1.0035x over previous
import jax
import jax.numpy as jnp
from jax.experimental import pallas as pl
from jax.experimental.pallas import tpu as pltpu

_NUM_ATTRIB = 85
_AW = (30.0, 62.0, 59.0)
_AH = (61.0, 45.0, 119.0)
_ROWS_PER_IMG = 19200
_BLK_ROWS = 9600
_SC_IMGS = 4
_TC_IMGS = 4


def _floordiv_f32(x, d):
    return jnp.floor((x + 0.5) * (1.0 / d))


def _decode(v, i):
    a = jax.lax.broadcasted_iota(jnp.int32, (1, _NUM_ATTRIB), 1)
    n = jnp.float32(i * _BLK_ROWS) + jax.lax.broadcasted_iota(
        jnp.int32, (_BLK_ROWS, 1), 0).astype(jnp.float32)
    pos = _floordiv_f32(n, 3.0)
    j = n - 3.0 * pos
    gy = _floordiv_f32(pos, 80.0)
    gx = pos - 80.0 * gy
    is_sig = a >= 4
    e = jnp.exp(jnp.where(is_sig, -v, v))
    sig = 1.0 / (1.0 + e)
    wsel = jnp.where(j == 0.0, _AW[0], jnp.where(j == 1.0, _AW[1], _AW[2]))
    hsel = jnp.where(j == 0.0, _AH[0], jnp.where(j == 1.0, _AH[1], _AH[2]))
    dim = jnp.where(a == 2, wsel, hsel)
    g = jnp.where(a == 0, gx, gy)
    lin = jnp.where((a == 2) | (a == 3), e * dim, (v + g) * 16.0)
    return jnp.where(is_sig, sig, lin)


def _body1(x_ref, o_ref):
    o_ref[0] = _decode(x_ref[0], pl.program_id(1))


def _body2(scale_ref, x_ref, prev_ref, o_ref):
    del prev_ref
    o_ref[0] = _decode(x_ref[0] * scale_ref[0, 0], pl.program_id(1))


def kernel(pred_map, num_imgs, level_idx):
    del level_idx
    ni = pred_map.shape[0]
    scale = jnp.asarray(num_imgs, jnp.float32) / ni
    y0 = jax.lax.optimization_barrier(pred_map[:_SC_IMGS]).reshape(
        _SC_IMGS, _ROWS_PER_IMG, _NUM_ATTRIB)
    y1 = pred_map[_SC_IMGS:].reshape(
        _TC_IMGS, _ROWS_PER_IMG, _NUM_ATTRIB) * scale
    blk = (1, _BLK_ROWS, _NUM_ATTRIB)
    out_sd = jax.ShapeDtypeStruct((ni, _ROWS_PER_IMG, _NUM_ATTRIB),
                                  jnp.float32)
    o1 = pl.pallas_call(
        _body1,
        grid=(_TC_IMGS, _ROWS_PER_IMG // _BLK_ROWS),
        in_specs=[pl.BlockSpec(blk, lambda b, i: (b, i, 0))],
        out_specs=pl.BlockSpec(blk, lambda b, i: (b + _SC_IMGS, i, 0)),
        out_shape=out_sd,
    )(y1)
    o2 = pl.pallas_call(
        _body2,
        grid=(_SC_IMGS, _ROWS_PER_IMG // _BLK_ROWS),
        in_specs=[
            pl.BlockSpec(memory_space=pltpu.SMEM),
            pl.BlockSpec(blk, lambda b, i: (b, i, 0)),
            pl.BlockSpec(memory_space=pl.ANY),
        ],
        out_specs=pl.BlockSpec(blk, lambda b, i: (b, i, 0)),
        out_shape=out_sd,
        input_output_aliases={2: 0},
    )(scale.reshape(1, 1), y0, o1)
    return o2
